# Initial kernel scaffold; baseline (speedup 1.0000x reference)
import jax
import jax.numpy as jnp
from jax.experimental import pallas as pl


def kernel(x, edge_index, batch, u, W1, b1, gn1_w, gn1_b, gn1_ms, W2, b2, gn2_w, gn2_b, gn2_ms, W3, b3, gn3_w, gn3_b, gn3_ms, Wlin, blin):
    raise NotImplementedError("write your pallas kernel here")


# speedup vs baseline: 11.1639x; 11.1639x over previous
import functools

import jax
import jax.numpy as jnp
from jax import lax
from jax.experimental import pallas as pl
from jax.experimental.pallas import tpu as pltpu
from jax.experimental.pallas import tpu_sc as plsc

NC = 2
NS = 16
NW = NC * NS
K = 128
ZR = 64


def _ceil_div(a, b):
    return (a + b - 1) // b


def _sc_mesh():
    return plsc.VectorSubcoreMesh(core_axis_name="c", subcore_axis_name="s")


_SC_PARAMS = pltpu.CompilerParams(use_tc_tiling_on_sc=False)


def _sc_degree(dst3, acc_rows):
    nchunk = dst3.shape[1]
    rpt = acc_rows // NS

    @functools.partial(
        pl.kernel,
        out_type=jax.ShapeDtypeStruct((NC, acc_rows, 16), jnp.float32),
        mesh=_sc_mesh(),
        scratch_types=[
            pltpu.VMEM((nchunk, K), jnp.int32),
            pltpu.VMEM((K, 16), jnp.float32),
            pltpu.VMEM((ZR, 16), jnp.float32),
            pltpu.VMEM_SHARED((acc_rows, 16), jnp.float32),
        ],
        compiler_params=_SC_PARAMS,
    )
    def deg_kernel(dst_hbm, out_hbm, dst_v, ones_v, zero_v, acc_sh):
        cid = lax.axis_index("c")
        sid = lax.axis_index("s")
        wid = cid * NS + sid
        one = jnp.full((16,), 1.0, jnp.float32)
        zro = jnp.zeros((16,), jnp.float32)

        @pl.loop(0, K)
        def _(r):
            ones_v[r, :] = one

        @pl.loop(0, ZR)
        def _(r):
            zero_v[r, :] = zro

        @pl.loop(0, rpt // ZR)
        def _(z):
            pltpu.sync_copy(zero_v, acc_sh.at[pl.ds(sid * rpt + z * ZR, ZR)])

        plsc.subcore_barrier()
        pltpu.sync_copy(dst_hbm.at[wid], dst_v)

        @pl.loop(0, nchunk)
        def _(j):
            pltpu.sync_copy(ones_v, acc_sh.at[dst_v.at[j]], add=True)

        plsc.subcore_barrier()
        pltpu.sync_copy(acc_sh.at[pl.ds(sid * rpt, rpt)],
                        out_hbm.at[cid, pl.ds(sid * rpt, rpt)])

    return deg_kernel(dst3)


def _sc_edge_agg(hp, src3, dst3, acc_rows):
    nchunk = src3.shape[1]
    C = hp.shape[1]
    rpt = acc_rows // NS

    @functools.partial(
        pl.kernel,
        out_type=jax.ShapeDtypeStruct((NC, acc_rows, C), jnp.float32),
        mesh=_sc_mesh(),
        scratch_types=[
            pltpu.VMEM((nchunk, K), jnp.int32),
            pltpu.VMEM((nchunk, K), jnp.int32),
            pltpu.VMEM((K, C), jnp.float32),
            pltpu.VMEM((K, C), jnp.float32),
            pltpu.VMEM((ZR, C), jnp.float32),
            pltpu.VMEM_SHARED((acc_rows, C), jnp.float32),
            pltpu.SemaphoreType.DMA,
            pltpu.SemaphoreType.DMA,
        ],
        compiler_params=_SC_PARAMS,
    )
    def agg_kernel(hp_hbm, src_hbm, dst_hbm, out_hbm,
                   src_v, dst_v, rows_a, rows_b, zero_v, acc_sh, sem_a, sem_b):
        cid = lax.axis_index("c")
        sid = lax.axis_index("s")
        wid = cid * NS + sid
        zro = jnp.zeros((16,), jnp.float32)

        @pl.loop(0, ZR)
        def _(r):
            @pl.loop(0, C // 16)
            def _(i):
                zero_v[r, pl.ds(i * 16, 16)] = zro

        @pl.loop(0, rpt // ZR)
        def _(z):
            pltpu.sync_copy(zero_v, acc_sh.at[pl.ds(sid * rpt + z * ZR, ZR)])

        plsc.subcore_barrier()
        pltpu.sync_copy(src_hbm.at[wid], src_v)
        pltpu.sync_copy(dst_hbm.at[wid], dst_v)

        pltpu.async_copy(hp_hbm.at[src_v.at[0]], rows_a, sem_a)

        @pl.loop(0, nchunk // 2)
        def _(jj):
            j = jj * 2
            pltpu.make_async_copy(hp_hbm.at[src_v.at[j]], rows_a, sem_a).wait()
            pltpu.async_copy(hp_hbm.at[src_v.at[j + 1]], rows_b, sem_b)
            pltpu.sync_copy(rows_a, acc_sh.at[dst_v.at[j]], add=True)
            pltpu.make_async_copy(hp_hbm.at[src_v.at[j + 1]], rows_b,
                                  sem_b).wait()

            @pl.when(j + 2 < nchunk)
            def _():
                pltpu.async_copy(hp_hbm.at[src_v.at[j + 2]], rows_a, sem_a)
            pltpu.sync_copy(rows_b, acc_sh.at[dst_v.at[j + 1]], add=True)

        plsc.subcore_barrier()
        pltpu.sync_copy(acc_sh.at[pl.ds(sid * rpt, rpt)],
                        out_hbm.at[cid, pl.ds(sid * rpt, rpt)])

    return agg_kernel(hp, src3, dst3)


def _tc_mm(x, W):
    n, _ = x.shape
    co = W.shape[1]

    def body(x_ref, w_ref, o_ref):
        o_ref[...] = jnp.dot(x_ref[...], w_ref[...],
                             preferred_element_type=jnp.float32,
                             precision=lax.Precision.HIGHEST)

    return pl.pallas_call(
        body, out_shape=jax.ShapeDtypeStruct((n, co), jnp.float32))(x, W)


def _tc_prep(degp, h1):
    n = h1.shape[0]

    def body(degp_ref, h1_ref, dinv_ref, hp_ref):
        deg = (degp_ref[0, 0:n, 0:1] + degp_ref[1, 0:n, 0:1]) + 1.0
        dinv = lax.rsqrt(jnp.maximum(deg, 1.0))
        dinv_ref[...] = dinv
        hp_ref[...] = h1_ref[...] * dinv

    return pl.pallas_call(
        body,
        out_shape=(jax.ShapeDtypeStruct((n, 1), jnp.float32),
                   jax.ShapeDtypeStruct(h1.shape, jnp.float32)),
    )(degp, h1)


RB = 2000


def _dotT(a, b):
    return lax.dot_general(a, b, (((0,), (0,)), ((), ())),
                           preferred_element_type=jnp.float32,
                           precision=lax.Precision.HIGHEST)


def _dot(a, b):
    return jnp.dot(a, b, preferred_element_type=jnp.float32,
                   precision=lax.Precision.HIGHEST)


def _combine_chunks(part_refs, hp_refs, dinv):
    ts = [dinv * (p[0] + p[1] + h[...]) for p, h in zip(part_refs, hp_refs)]
    return ts[0] if len(ts) == 1 else jnp.concatenate(ts, axis=1)


def _mt_block(batch_col_blk, g):
    return (lax.broadcasted_iota(jnp.int32, (RB, g), 1)
            == batch_col_blk).astype(jnp.float32)


def _row_specs(nc, C_chunks):
    specs = [pl.BlockSpec((2, RB, c), lambda i: (0, i, 0)) for c in C_chunks]
    specs += [pl.BlockSpec((RB, c), lambda i: (i, 0)) for c in C_chunks]
    specs += [pl.BlockSpec((RB, 1), lambda i: (i, 0)),
              pl.BlockSpec((RB, 1), lambda i: (i, 0))]
    return specs


def _tc_stats(parts, hps, dinv, b, batch_col, g):
    n = hps[0].shape[0]
    nc = len(parts)
    C = sum(h.shape[1] for h in hps)

    def body(*refs):
        part_refs = refs[:nc]
        hp_refs = refs[nc:2 * nc]
        dinv_ref, bc_ref, b_ref, s1_ref, s2_ref, cnt_ref = refs[2 * nc:]
        t = _combine_chunks(part_refs, hp_refs, dinv_ref[...]) + b_ref[...]
        mt = _mt_block(bc_ref[...], g)

        @pl.when(pl.program_id(0) == 0)
        def _():
            s1_ref[...] = jnp.zeros_like(s1_ref)
            s2_ref[...] = jnp.zeros_like(s2_ref)
            cnt_ref[...] = jnp.zeros_like(cnt_ref)

        s1_ref[...] += _dotT(mt, t)
        s2_ref[...] += _dotT(mt, t * t)
        cnt_ref[...] += _dotT(mt, jnp.ones((RB, 1), jnp.float32))

    in_specs = _row_specs(nc, [h.shape[1] for h in hps])
    in_specs.append(pl.BlockSpec(b.shape, lambda i: (0, 0)))
    return pl.pallas_call(
        body,
        grid=(n // RB,),
        in_specs=in_specs,
        out_specs=(pl.BlockSpec((g, C), lambda i: (0, 0)),
                   pl.BlockSpec((g, C), lambda i: (0, 0)),
                   pl.BlockSpec((g, 1), lambda i: (0, 0))),
        out_shape=(jax.ShapeDtypeStruct((g, C), jnp.float32),
                   jax.ShapeDtypeStruct((g, C), jnp.float32),
                   jax.ShapeDtypeStruct((g, 1), jnp.float32)),
    )(*parts, *hps, dinv, batch_col, b)


def _gnorm_coeffs(s1, s2, cntp, gw, gb, gms):
    cnt = jnp.maximum(cntp, 1.0)
    mean = s1 / cnt
    mt = mean * gms
    var = s2 / cnt - 2.0 * mt * mean + mt * mt
    std = jnp.sqrt(var + 1e-5)
    A = gw / std
    Cc = gb - A * mt
    return A, Cc


def _tc_apply(parts, hps, dinv, stats, b, gw, gb, gms, Wn_blocks,
              batch_col, g):
    n = hps[0].shape[0]
    nc = len(parts)
    s1, s2, cntp = stats

    def body(*refs):
        part_refs = refs[:nc]
        hp_refs = refs[nc:2 * nc]
        (dinv_ref, bc_ref, s1_ref, s2_ref, cnt_ref, b_ref, gw_ref, gb_ref,
         gms_ref) = refs[2 * nc:2 * nc + 9]
        w_refs = refs[2 * nc + 9:2 * nc + 9 + len(Wn_blocks)]
        o_refs = refs[2 * nc + 9 + len(Wn_blocks):]
        dinv = dinv_ref[...]
        t = _combine_chunks(part_refs, hp_refs, dinv) + b_ref[...]
        A, Cc = _gnorm_coeffs(s1_ref[...], s2_ref[...], cnt_ref[...],
                              gw_ref[...], gb_ref[...], gms_ref[...])
        mt = _mt_block(bc_ref[...], g)
        y = jnp.maximum(_dot(mt, A) * t + _dot(mt, Cc), 0.0)
        for w_ref, o_ref in zip(w_refs, o_refs):
            o_ref[...] = _dot(y, w_ref[...]) * dinv

    C = sum(h.shape[1] for h in hps)
    in_specs = _row_specs(nc, [h.shape[1] for h in hps])
    in_specs += [pl.BlockSpec((g, C), lambda i: (0, 0)),
                 pl.BlockSpec((g, C), lambda i: (0, 0)),
                 pl.BlockSpec((g, 1), lambda i: (0, 0)),
                 pl.BlockSpec(b.shape, lambda i: (0, 0)),
                 pl.BlockSpec(gw.shape, lambda i: (0, 0)),
                 pl.BlockSpec(gb.shape, lambda i: (0, 0)),
                 pl.BlockSpec(gms.shape, lambda i: (0, 0))]
    in_specs += [pl.BlockSpec(w.shape, lambda i: (0, 0)) for w in Wn_blocks]
    return pl.pallas_call(
        body,
        grid=(n // RB,),
        in_specs=in_specs,
        out_specs=tuple(pl.BlockSpec((RB, w.shape[1]), lambda i: (i, 0))
                        for w in Wn_blocks),
        out_shape=tuple(jax.ShapeDtypeStruct((n, w.shape[1]), jnp.float32)
                        for w in Wn_blocks),
    )(*parts, *hps, dinv, batch_col, s1, s2, cntp, b, gw, gb, gms, *Wn_blocks)


def _tc_layer(parts, hps, dinv, b, gw, gb, gms, Wn_blocks, batch_col, g):
    stats = _tc_stats(parts, hps, dinv, b, batch_col, g)
    return _tc_apply(parts, hps, dinv, stats, b, gw, gb, gms, Wn_blocks,
                     batch_col, g)


def _tc_pool(parts, hps, dinv, stats, b, gw, gb, gms, batch_col, g):
    n = hps[0].shape[0]
    nc = len(parts)
    s1, s2, cntp = stats
    C = sum(h.shape[1] for h in hps)

    def body(*refs):
        part_refs = refs[:nc]
        hp_refs = refs[nc:2 * nc]
        (dinv_ref, bc_ref, s1_ref, s2_ref, cnt_ref, b_ref, gw_ref, gb_ref,
         gms_ref, o_ref) = refs[2 * nc:]
        t = _combine_chunks(part_refs, hp_refs, dinv_ref[...]) + b_ref[...]
        A, Cc = _gnorm_coeffs(s1_ref[...], s2_ref[...], cnt_ref[...],
                              gw_ref[...], gb_ref[...], gms_ref[...])
        mt = _mt_block(bc_ref[...], g)
        y = jnp.maximum(_dot(mt, A) * t + _dot(mt, Cc), 0.0)

        @pl.when(pl.program_id(0) == 0)
        def _():
            o_ref[...] = jnp.zeros_like(o_ref)

        o_ref[...] += _dotT(mt, y)

    in_specs = _row_specs(nc, [h.shape[1] for h in hps])
    in_specs += [pl.BlockSpec((g, C), lambda i: (0, 0)),
                 pl.BlockSpec((g, C), lambda i: (0, 0)),
                 pl.BlockSpec((g, 1), lambda i: (0, 0)),
                 pl.BlockSpec(b.shape, lambda i: (0, 0)),
                 pl.BlockSpec(gw.shape, lambda i: (0, 0)),
                 pl.BlockSpec(gb.shape, lambda i: (0, 0)),
                 pl.BlockSpec(gms.shape, lambda i: (0, 0))]
    return pl.pallas_call(
        body,
        grid=(n // RB,),
        in_specs=in_specs,
        out_specs=pl.BlockSpec((g, C), lambda i: (0, 0)),
        out_shape=jax.ShapeDtypeStruct((g, C), jnp.float32),
    )(*parts, *hps, dinv, batch_col, s1, s2, cntp, b, gw, gb, gms)


def _tc_out(pool, cntp, u, Wlin_h, Wlin_u, blin, g):
    def body(pool_ref, cnt_ref, u_ref, wh_ref, wu_ref, bl_ref, o_ref):
        cnt = jnp.maximum(cnt_ref[...], 1.0)
        pooled = pool_ref[...] / cnt
        o_ref[...] = (_dot(pooled, wh_ref[...]) + _dot(u_ref[...], wu_ref[...])
                      + bl_ref[...])

    return pl.pallas_call(
        body, out_shape=jax.ShapeDtypeStruct((g, 1), jnp.float32),
    )(pool, cntp, u, Wlin_h, Wlin_u, blin)


def kernel(x, edge_index, batch, u, W1, b1, gn1_w, gn1_b, gn1_ms,
           W2, b2, gn2_w, gn2_b, gn2_ms, W3, b3, gn3_w, gn3_b, gn3_ms,
           Wlin, blin):
    n = x.shape[0]
    e = edge_index.shape[1]
    g = u.shape[0]

    nchunk = _ceil_div(e, NW * K)
    nchunk += nchunk % 2
    epad = NW * nchunk * K - e
    acc_rows = _ceil_div(n + 1, NS * ZR) * NS * ZR

    src = edge_index[0].astype(jnp.int32)
    dst = edge_index[1].astype(jnp.int32)
    src3 = jnp.concatenate(
        [src, jnp.zeros((epad,), jnp.int32)]).reshape(NW, nchunk, K)
    dst3 = jnp.concatenate(
        [dst, jnp.full((epad,), n, jnp.int32)]).reshape(NW, nchunk, K)

    batch_col = batch.astype(jnp.int32).reshape(n, 1)

    degp = _sc_degree(dst3, acc_rows)
    h1 = _tc_mm(x, W1)
    dinv, hp = _tc_prep(degp, h1)

    part1 = _sc_edge_agg(hp, src3, dst3, acc_rows)
    (hp,) = _tc_layer([part1], [hp], dinv, b1.reshape(1, -1),
                      gn1_w.reshape(1, -1), gn1_b.reshape(1, -1),
                      gn1_ms.reshape(1, -1), [W2], batch_col, g)

    part2 = _sc_edge_agg(hp, src3, dst3, acc_rows)
    ch3 = W3.shape[1]
    hp3 = _tc_layer([part2], [hp], dinv, b2.reshape(1, -1),
                    gn2_w.reshape(1, -1), gn2_b.reshape(1, -1),
                    gn2_ms.reshape(1, -1), [W3[:, :ch3 // 2], W3[:, ch3 // 2:]],
                    batch_col, g)

    parts3 = [_sc_edge_agg(h, src3, dst3, acc_rows) for h in hp3]
    b3r = b3.reshape(1, -1)
    stats3 = _tc_stats(parts3, list(hp3), dinv, b3r, batch_col, g)
    pool = _tc_pool(parts3, list(hp3), dinv, stats3, b3r,
                    gn3_w.reshape(1, -1), gn3_b.reshape(1, -1),
                    gn3_ms.reshape(1, -1), batch_col, g)
    out = _tc_out(pool, stats3[2], u, Wlin[:ch3], Wlin[ch3:],
                  blin.reshape(1, 1), g)
    return out


# speedup vs baseline: 12.0027x; 1.0751x over previous
import functools

import jax
import jax.numpy as jnp
from jax import lax
from jax.experimental import pallas as pl
from jax.experimental.pallas import tpu as pltpu
from jax.experimental.pallas import tpu_sc as plsc

NC = 2
NS = 16
NW = NC * NS
K = 128
ZR = 64
NB = 4


def _ceil_div(a, b):
    return (a + b - 1) // b


def _sc_mesh():
    return plsc.VectorSubcoreMesh(core_axis_name="c", subcore_axis_name="s")


_SC_PARAMS = pltpu.CompilerParams(use_tc_tiling_on_sc=False)


def _sc_degree(dst3, acc_rows):
    nchunk = dst3.shape[1]
    rpt = acc_rows // NS

    @functools.partial(
        pl.kernel,
        out_type=jax.ShapeDtypeStruct((NC, acc_rows, 16), jnp.float32),
        mesh=_sc_mesh(),
        scratch_types=[
            pltpu.VMEM((nchunk, K), jnp.int32),
            pltpu.VMEM((K, 16), jnp.float32),
            pltpu.VMEM((ZR, 16), jnp.float32),
            pltpu.VMEM_SHARED((acc_rows, 16), jnp.float32),
        ],
        compiler_params=_SC_PARAMS,
    )
    def deg_kernel(dst_hbm, out_hbm, dst_v, ones_v, zero_v, acc_sh):
        cid = lax.axis_index("c")
        sid = lax.axis_index("s")
        wid = cid * NS + sid
        one = jnp.full((16,), 1.0, jnp.float32)
        zro = jnp.zeros((16,), jnp.float32)

        @pl.loop(0, K)
        def _(r):
            ones_v[r, :] = one

        @pl.loop(0, ZR)
        def _(r):
            zero_v[r, :] = zro

        @pl.loop(0, rpt // ZR)
        def _(z):
            pltpu.sync_copy(zero_v, acc_sh.at[pl.ds(sid * rpt + z * ZR, ZR)])

        plsc.subcore_barrier()
        pltpu.sync_copy(dst_hbm.at[wid], dst_v)

        @pl.loop(0, nchunk)
        def _(j):
            pltpu.sync_copy(ones_v, acc_sh.at[dst_v.at[j]], add=True)

        plsc.subcore_barrier()
        pltpu.sync_copy(acc_sh.at[pl.ds(sid * rpt, rpt)],
                        out_hbm.at[cid, pl.ds(sid * rpt, rpt)])

    return deg_kernel(dst3)


def _sc_edge_agg(hp, src3, dst3, acc_rows):
    nchunk = src3.shape[1]
    C = hp.shape[1]
    rpt = acc_rows // NS

    @functools.partial(
        pl.kernel,
        out_type=jax.ShapeDtypeStruct((NC, acc_rows, C), jnp.float32),
        mesh=_sc_mesh(),
        scratch_types=[
            pltpu.VMEM((nchunk, K), jnp.int32),
            pltpu.VMEM((nchunk, K), jnp.int32),
            pltpu.VMEM((NB, K, C), jnp.float32),
            pltpu.VMEM((ZR, C), jnp.float32),
            pltpu.VMEM_SHARED((acc_rows, C), jnp.float32),
            [pltpu.SemaphoreType.DMA] * NB,
            [pltpu.SemaphoreType.DMA] * NB,
        ],
        compiler_params=_SC_PARAMS,
    )
    def agg_kernel(hp_hbm, src_hbm, dst_hbm, out_hbm,
                   src_v, dst_v, rows_v, zero_v, acc_sh, sem_g, sem_s):
        cid = lax.axis_index("c")
        sid = lax.axis_index("s")
        wid = cid * NS + sid
        zro = jnp.zeros((16,), jnp.float32)

        @pl.loop(0, ZR)
        def _(r):
            @pl.loop(0, C // 16)
            def _(i):
                zero_v[r, pl.ds(i * 16, 16)] = zro

        @pl.loop(0, rpt // ZR)
        def _(z):
            pltpu.sync_copy(zero_v, acc_sh.at[pl.ds(sid * rpt + z * ZR, ZR)])

        plsc.subcore_barrier()
        pltpu.sync_copy(src_hbm.at[wid], src_v)
        pltpu.sync_copy(dst_hbm.at[wid], dst_v)

        for b in range(NB):
            pltpu.async_copy(hp_hbm.at[src_v.at[b]], rows_v.at[b], sem_g[b])

        @pl.loop(0, nchunk // NB)
        def _(jj):
            j = jj * NB
            for b in range(NB):
                pltpu.make_async_copy(hp_hbm.at[src_v.at[j + b]],
                                      rows_v.at[b], sem_g[b]).wait()
                pltpu.async_copy(rows_v.at[b], acc_sh.at[dst_v.at[j + b]],
                                 sem_s[b], add=True)
            for b in range(NB):
                m = j + NB + b

                @pl.when(m < nchunk)
                def _():
                    pltpu.make_async_copy(rows_v.at[b],
                                          acc_sh.at[dst_v.at[j + b]],
                                          sem_s[b]).wait()
                    pltpu.async_copy(hp_hbm.at[src_v.at[m]], rows_v.at[b],
                                     sem_g[b])

            @pl.when(j + NB >= nchunk)
            def _():
                for b in range(NB):
                    pltpu.make_async_copy(rows_v.at[b],
                                          acc_sh.at[dst_v.at[j + b]],
                                          sem_s[b]).wait()

        plsc.subcore_barrier()
        pltpu.sync_copy(acc_sh.at[pl.ds(sid * rpt, rpt)],
                        out_hbm.at[cid, pl.ds(sid * rpt, rpt)])

    return agg_kernel(hp, src3, dst3)


def _tc_mm(x, W):
    n, _ = x.shape
    co = W.shape[1]

    def body(x_ref, w_ref, o_ref):
        o_ref[...] = jnp.dot(x_ref[...], w_ref[...],
                             preferred_element_type=jnp.float32,
                             precision=lax.Precision.HIGHEST)

    return pl.pallas_call(
        body, out_shape=jax.ShapeDtypeStruct((n, co), jnp.float32))(x, W)


def _tc_prep(degp, h1):
    n = h1.shape[0]

    def body(degp_ref, h1_ref, dinv_ref, hp_ref):
        deg = (degp_ref[0, 0:n, 0:1] + degp_ref[1, 0:n, 0:1]) + 1.0
        dinv = lax.rsqrt(jnp.maximum(deg, 1.0))
        dinv_ref[...] = dinv
        hp_ref[...] = h1_ref[...] * dinv

    return pl.pallas_call(
        body,
        out_shape=(jax.ShapeDtypeStruct((n, 1), jnp.float32),
                   jax.ShapeDtypeStruct(h1.shape, jnp.float32)),
    )(degp, h1)


RB = 2000


def _dotT(a, b):
    return lax.dot_general(a, b, (((0,), (0,)), ((), ())),
                           preferred_element_type=jnp.float32,
                           precision=lax.Precision.HIGHEST)


def _dot(a, b):
    return jnp.dot(a, b, preferred_element_type=jnp.float32,
                   precision=lax.Precision.HIGHEST)


def _combine_chunks(part_refs, hp_refs, dinv):
    ts = [dinv * (p[0] + p[1] + h[...]) for p, h in zip(part_refs, hp_refs)]
    return ts[0] if len(ts) == 1 else jnp.concatenate(ts, axis=1)


def _mt_block(batch_col_blk, g):
    return (lax.broadcasted_iota(jnp.int32, (RB, g), 1)
            == batch_col_blk).astype(jnp.float32)


def _row_specs(nc, C_chunks):
    specs = [pl.BlockSpec((2, RB, c), lambda i: (0, i, 0)) for c in C_chunks]
    specs += [pl.BlockSpec((RB, c), lambda i: (i, 0)) for c in C_chunks]
    specs += [pl.BlockSpec((RB, 1), lambda i: (i, 0)),
              pl.BlockSpec((RB, 1), lambda i: (i, 0))]
    return specs


def _tc_stats(parts, hps, dinv, b, batch_col, g):
    n = hps[0].shape[0]
    nc = len(parts)
    C = sum(h.shape[1] for h in hps)

    def body(*refs):
        part_refs = refs[:nc]
        hp_refs = refs[nc:2 * nc]
        dinv_ref, bc_ref, b_ref, s1_ref, s2_ref, cnt_ref = refs[2 * nc:]
        t = _combine_chunks(part_refs, hp_refs, dinv_ref[...]) + b_ref[...]
        mt = _mt_block(bc_ref[...], g)

        @pl.when(pl.program_id(0) == 0)
        def _():
            s1_ref[...] = jnp.zeros_like(s1_ref)
            s2_ref[...] = jnp.zeros_like(s2_ref)
            cnt_ref[...] = jnp.zeros_like(cnt_ref)

        s1_ref[...] += _dotT(mt, t)
        s2_ref[...] += _dotT(mt, t * t)
        cnt_ref[...] += _dotT(mt, jnp.ones((RB, 1), jnp.float32))

    in_specs = _row_specs(nc, [h.shape[1] for h in hps])
    in_specs.append(pl.BlockSpec(b.shape, lambda i: (0, 0)))
    return pl.pallas_call(
        body,
        grid=(n // RB,),
        in_specs=in_specs,
        out_specs=(pl.BlockSpec((g, C), lambda i: (0, 0)),
                   pl.BlockSpec((g, C), lambda i: (0, 0)),
                   pl.BlockSpec((g, 1), lambda i: (0, 0))),
        out_shape=(jax.ShapeDtypeStruct((g, C), jnp.float32),
                   jax.ShapeDtypeStruct((g, C), jnp.float32),
                   jax.ShapeDtypeStruct((g, 1), jnp.float32)),
    )(*parts, *hps, dinv, batch_col, b)


def _gnorm_coeffs(s1, s2, cntp, gw, gb, gms):
    cnt = jnp.maximum(cntp, 1.0)
    mean = s1 / cnt
    mt = mean * gms
    var = s2 / cnt - 2.0 * mt * mean + mt * mt
    std = jnp.sqrt(var + 1e-5)
    A = gw / std
    Cc = gb - A * mt
    return A, Cc


def _tc_apply(parts, hps, dinv, stats, b, gw, gb, gms, Wn_blocks,
              batch_col, g):
    n = hps[0].shape[0]
    nc = len(parts)
    s1, s2, cntp = stats

    def body(*refs):
        part_refs = refs[:nc]
        hp_refs = refs[nc:2 * nc]
        (dinv_ref, bc_ref, s1_ref, s2_ref, cnt_ref, b_ref, gw_ref, gb_ref,
         gms_ref) = refs[2 * nc:2 * nc + 9]
        w_refs = refs[2 * nc + 9:2 * nc + 9 + len(Wn_blocks)]
        o_refs = refs[2 * nc + 9 + len(Wn_blocks):]
        dinv = dinv_ref[...]
        t = _combine_chunks(part_refs, hp_refs, dinv) + b_ref[...]
        A, Cc = _gnorm_coeffs(s1_ref[...], s2_ref[...], cnt_ref[...],
                              gw_ref[...], gb_ref[...], gms_ref[...])
        mt = _mt_block(bc_ref[...], g)
        y = jnp.maximum(_dot(mt, A) * t + _dot(mt, Cc), 0.0)
        for w_ref, o_ref in zip(w_refs, o_refs):
            o_ref[...] = _dot(y, w_ref[...]) * dinv

    C = sum(h.shape[1] for h in hps)
    in_specs = _row_specs(nc, [h.shape[1] for h in hps])
    in_specs += [pl.BlockSpec((g, C), lambda i: (0, 0)),
                 pl.BlockSpec((g, C), lambda i: (0, 0)),
                 pl.BlockSpec((g, 1), lambda i: (0, 0)),
                 pl.BlockSpec(b.shape, lambda i: (0, 0)),
                 pl.BlockSpec(gw.shape, lambda i: (0, 0)),
                 pl.BlockSpec(gb.shape, lambda i: (0, 0)),
                 pl.BlockSpec(gms.shape, lambda i: (0, 0))]
    in_specs += [pl.BlockSpec(w.shape, lambda i: (0, 0)) for w in Wn_blocks]
    return pl.pallas_call(
        body,
        grid=(n // RB,),
        in_specs=in_specs,
        out_specs=tuple(pl.BlockSpec((RB, w.shape[1]), lambda i: (i, 0))
                        for w in Wn_blocks),
        out_shape=tuple(jax.ShapeDtypeStruct((n, w.shape[1]), jnp.float32)
                        for w in Wn_blocks),
    )(*parts, *hps, dinv, batch_col, s1, s2, cntp, b, gw, gb, gms, *Wn_blocks)


def _tc_layer(parts, hps, dinv, b, gw, gb, gms, Wn_blocks, batch_col, g):
    stats = _tc_stats(parts, hps, dinv, b, batch_col, g)
    return _tc_apply(parts, hps, dinv, stats, b, gw, gb, gms, Wn_blocks,
                     batch_col, g)


def _tc_pool(parts, hps, dinv, stats, b, gw, gb, gms, batch_col, g):
    n = hps[0].shape[0]
    nc = len(parts)
    s1, s2, cntp = stats
    C = sum(h.shape[1] for h in hps)

    def body(*refs):
        part_refs = refs[:nc]
        hp_refs = refs[nc:2 * nc]
        (dinv_ref, bc_ref, s1_ref, s2_ref, cnt_ref, b_ref, gw_ref, gb_ref,
         gms_ref, o_ref) = refs[2 * nc:]
        t = _combine_chunks(part_refs, hp_refs, dinv_ref[...]) + b_ref[...]
        A, Cc = _gnorm_coeffs(s1_ref[...], s2_ref[...], cnt_ref[...],
                              gw_ref[...], gb_ref[...], gms_ref[...])
        mt = _mt_block(bc_ref[...], g)
        y = jnp.maximum(_dot(mt, A) * t + _dot(mt, Cc), 0.0)

        @pl.when(pl.program_id(0) == 0)
        def _():
            o_ref[...] = jnp.zeros_like(o_ref)

        o_ref[...] += _dotT(mt, y)

    in_specs = _row_specs(nc, [h.shape[1] for h in hps])
    in_specs += [pl.BlockSpec((g, C), lambda i: (0, 0)),
                 pl.BlockSpec((g, C), lambda i: (0, 0)),
                 pl.BlockSpec((g, 1), lambda i: (0, 0)),
                 pl.BlockSpec(b.shape, lambda i: (0, 0)),
                 pl.BlockSpec(gw.shape, lambda i: (0, 0)),
                 pl.BlockSpec(gb.shape, lambda i: (0, 0)),
                 pl.BlockSpec(gms.shape, lambda i: (0, 0))]
    return pl.pallas_call(
        body,
        grid=(n // RB,),
        in_specs=in_specs,
        out_specs=pl.BlockSpec((g, C), lambda i: (0, 0)),
        out_shape=jax.ShapeDtypeStruct((g, C), jnp.float32),
    )(*parts, *hps, dinv, batch_col, s1, s2, cntp, b, gw, gb, gms)


def _tc_out(pool, cntp, u, Wlin_h, Wlin_u, blin, g):
    def body(pool_ref, cnt_ref, u_ref, wh_ref, wu_ref, bl_ref, o_ref):
        cnt = jnp.maximum(cnt_ref[...], 1.0)
        pooled = pool_ref[...] / cnt
        o_ref[...] = (_dot(pooled, wh_ref[...]) + _dot(u_ref[...], wu_ref[...])
                      + bl_ref[...])

    return pl.pallas_call(
        body, out_shape=jax.ShapeDtypeStruct((g, 1), jnp.float32),
    )(pool, cntp, u, Wlin_h, Wlin_u, blin)


def kernel(x, edge_index, batch, u, W1, b1, gn1_w, gn1_b, gn1_ms,
           W2, b2, gn2_w, gn2_b, gn2_ms, W3, b3, gn3_w, gn3_b, gn3_ms,
           Wlin, blin):
    n = x.shape[0]
    e = edge_index.shape[1]
    g = u.shape[0]

    nchunk = _ceil_div(e, NW * K)
    nchunk += (-nchunk) % NB
    epad = NW * nchunk * K - e
    acc_rows = _ceil_div(n + 1, NS * ZR) * NS * ZR

    src = edge_index[0].astype(jnp.int32)
    dst = edge_index[1].astype(jnp.int32)
    pad_dst = n + (jnp.arange(epad, dtype=jnp.int32) % (acc_rows - n))
    src3 = jnp.concatenate(
        [src, jnp.zeros((epad,), jnp.int32)]).reshape(NW, nchunk, K)
    dst3 = jnp.concatenate([dst, pad_dst]).reshape(NW, nchunk, K)

    batch_col = batch.astype(jnp.int32).reshape(n, 1)

    degp = _sc_degree(dst3, acc_rows)
    h1 = _tc_mm(x, W1)
    dinv, hp = _tc_prep(degp, h1)

    part1 = _sc_edge_agg(hp, src3, dst3, acc_rows)
    (hp,) = _tc_layer([part1], [hp], dinv, b1.reshape(1, -1),
                      gn1_w.reshape(1, -1), gn1_b.reshape(1, -1),
                      gn1_ms.reshape(1, -1), [W2], batch_col, g)

    part2 = _sc_edge_agg(hp, src3, dst3, acc_rows)
    ch3 = W3.shape[1]
    hp3 = _tc_layer([part2], [hp], dinv, b2.reshape(1, -1),
                    gn2_w.reshape(1, -1), gn2_b.reshape(1, -1),
                    gn2_ms.reshape(1, -1), [W3[:, :ch3 // 2], W3[:, ch3 // 2:]],
                    batch_col, g)

    parts3 = [_sc_edge_agg(h, src3, dst3, acc_rows) for h in hp3]
    b3r = b3.reshape(1, -1)
    stats3 = _tc_stats(parts3, list(hp3), dinv, b3r, batch_col, g)
    pool = _tc_pool(parts3, list(hp3), dinv, stats3, b3r,
                    gn3_w.reshape(1, -1), gn3_b.reshape(1, -1),
                    gn3_ms.reshape(1, -1), batch_col, g)
    out = _tc_out(pool, stats3[2], u, Wlin[:ch3], Wlin[ch3:],
                  blin.reshape(1, 1), g)
    return out
